# initial kernel scaffold (unmeasured)
import jax
import jax.numpy as jnp
from jax import lax
from jax.experimental import pallas as pl
from jax.experimental.pallas import tpu as pltpu

N_DEV = 16
LOG2P = 4
N_LAYERS = 3
N_SLOTS = N_LAYERS * LOG2P


def kernel(x, Win0, Wout0, Win1, Wout1, Win2, Wout2):
    b, d_shard = x.shape
    hidden = Win0.shape[1]

    def body(x_ref, win0_ref, wout0_ref, win1_ref, wout1_ref, win2_ref,
             wout2_ref, out_ref, acc_ref, recv_ref, send_sems, recv_sems):
        my = lax.axis_index("i")

        barrier_sem = pltpu.get_barrier_semaphore()
        for k in range(LOG2P):
            partner = my ^ (1 << k)
            pl.semaphore_signal(
                barrier_sem, inc=1,
                device_id=(partner,), device_id_type=pl.DeviceIdType.MESH,
            )
        pl.semaphore_wait(barrier_sem, LOG2P)

        wins = [win0_ref, win1_ref, win2_ref]
        wouts = [wout0_ref, wout1_ref, wout2_ref]

        xv = x_ref[...].astype(jnp.bfloat16)
        for l in range(N_LAYERS):
            w_in = wins[l][...].astype(jnp.bfloat16)
            acc_ref[...] = jnp.dot(xv, w_in,
                                   preferred_element_type=jnp.float32)

            for k in range(LOG2P):
                partner = my ^ (1 << k)
                slot = l * LOG2P + k
                rdma = pltpu.make_async_remote_copy(
                    src_ref=acc_ref,
                    dst_ref=recv_ref.at[slot],
                    send_sem=send_sems.at[slot],
                    recv_sem=recv_sems.at[slot],
                    device_id=(partner,),
                    device_id_type=pl.DeviceIdType.MESH,
                )
                rdma.start()
                rdma.wait()
                acc_ref[...] += recv_ref[slot]

            h = jnp.maximum(acc_ref[...], 0.0).astype(jnp.bfloat16)
            w_out = wouts[l][...].astype(jnp.bfloat16)
            xv = jnp.dot(h, w_out, preferred_element_type=jnp.float32)
            if l == N_LAYERS - 1:
                out_ref[...] = xv
            else:
                xv = xv.astype(jnp.bfloat16)

    return pl.pallas_call(
        body,
        out_shape=jax.ShapeDtypeStruct((b, d_shard), jnp.float32),
        in_specs=[pl.BlockSpec(memory_space=pltpu.VMEM)] * 7,
        out_specs=pl.BlockSpec(memory_space=pltpu.VMEM),
        scratch_shapes=[
            pltpu.VMEM((b, hidden), jnp.float32),
            pltpu.VMEM((N_SLOTS, b, hidden), jnp.float32),
            pltpu.SemaphoreType.DMA((N_SLOTS,)),
            pltpu.SemaphoreType.DMA((N_SLOTS,)),
        ],
        compiler_params=pltpu.CompilerParams(collective_id=0),
    )(x, Win0, Wout0, Win1, Wout1, Win2, Wout2)


# baseline (device time: 134317 ns/iter reference)
import jax
import jax.numpy as jnp
from jax import lax
from jax.experimental import pallas as pl
from jax.experimental.pallas import tpu as pltpu

N_DEV = 16
LOG2P = 4
N_LAYERS = 3
N_SLOTS = N_LAYERS * LOG2P


def kernel(x, Win0, Wout0, Win1, Wout1, Win2, Wout2):
    b, d_shard = x.shape
    hidden = Win0.shape[1]

    def body(x_ref, win0_ref, wout0_ref, win1_ref, wout1_ref, win2_ref,
             wout2_ref, out_ref, acc_ref, recv_ref, send_sems, recv_sems):
        my = lax.axis_index("i")

        barrier_sem = pltpu.get_barrier_semaphore()
        for k in range(LOG2P):
            partner = my ^ (1 << k)
            pl.semaphore_signal(
                barrier_sem, inc=1,
                device_id=(partner,), device_id_type=pl.DeviceIdType.MESH,
            )
        pl.semaphore_wait(barrier_sem, LOG2P)

        wins = [win0_ref, win1_ref, win2_ref]
        wouts = [wout0_ref, wout1_ref, wout2_ref]

        xv = x_ref[...].astype(jnp.bfloat16)
        for l in range(N_LAYERS):
            w_in = wins[l][...].astype(jnp.bfloat16)
            acc_ref[...] = jnp.dot(xv, w_in,
                                   preferred_element_type=jnp.float32)

            for k in range(LOG2P):
                partner = my ^ (1 << k)
                slot = l * LOG2P + k
                rdma = pltpu.make_async_remote_copy(
                    src_ref=acc_ref,
                    dst_ref=recv_ref.at[slot],
                    send_sem=send_sems.at[slot],
                    recv_sem=recv_sems.at[slot],
                    device_id=(partner,),
                    device_id_type=pl.DeviceIdType.MESH,
                )
                rdma.start()
                rdma.wait()
                acc_ref[...] += recv_ref[slot]

            h = jnp.maximum(acc_ref[...], 0.0).astype(jnp.bfloat16)
            w_out = wouts[l][...].astype(jnp.bfloat16)
            xv = jnp.dot(h, w_out, preferred_element_type=jnp.float32)
            if l == N_LAYERS - 1:
                out_ref[...] = xv
            else:
                xv = xv.astype(jnp.bfloat16)

    return pl.pallas_call(
        body,
        out_shape=jax.ShapeDtypeStruct((b, d_shard), jnp.float32),
        in_specs=[pl.BlockSpec(memory_space=pltpu.VMEM)] * 7,
        out_specs=pl.BlockSpec(memory_space=pltpu.VMEM),
        scratch_shapes=[
            pltpu.VMEM((b, hidden), jnp.float32),
            pltpu.VMEM((N_SLOTS, b, hidden), jnp.float32),
            pltpu.SemaphoreType.DMA((N_SLOTS,)),
            pltpu.SemaphoreType.DMA((N_SLOTS,)),
        ],
        compiler_params=pltpu.CompilerParams(
            collective_id=0,
            vmem_limit_bytes=100 * 1024 * 1024,
        ),
    )(x, Win0, Wout0, Win1, Wout1, Win2, Wout2)


# device time: 92079 ns/iter; 1.4587x vs baseline; 1.4587x over previous
import jax
import jax.numpy as jnp
from jax import lax
from jax.experimental import pallas as pl
from jax.experimental.pallas import tpu as pltpu

N_DEV = 16
LOG2P = 4
N_LAYERS = 3
N_SLOTS = N_LAYERS * LOG2P


def kernel(x, Win0, Wout0, Win1, Wout1, Win2, Wout2):
    b, d_shard = x.shape
    hidden = Win0.shape[1]

    def body(x_ref, win0_ref, wout0_ref, win1_ref, wout1_ref, win2_ref,
             wout2_ref, out_ref, acc_ref, sendb_ref, recv_ref, send_sems,
             recv_sems):
        my = lax.axis_index("i")

        barrier_sem = pltpu.get_barrier_semaphore()
        for k in range(LOG2P):
            partner = my ^ (1 << k)
            pl.semaphore_signal(
                barrier_sem, inc=1,
                device_id=(partner,), device_id_type=pl.DeviceIdType.MESH,
            )
        pl.semaphore_wait(barrier_sem, LOG2P)

        wins = [win0_ref, win1_ref, win2_ref]
        wouts = [wout0_ref, wout1_ref, wout2_ref]

        xv = x_ref[...].astype(jnp.bfloat16)
        for l in range(N_LAYERS):
            w_in = wins[l][...].astype(jnp.bfloat16)
            acc_ref[...] = jnp.dot(xv, w_in,
                                   preferred_element_type=jnp.float32)

            for k in range(LOG2P):
                partner = my ^ (1 << k)
                slot = l * LOG2P + k
                sendb_ref[...] = acc_ref[...].astype(jnp.bfloat16)
                rdma = pltpu.make_async_remote_copy(
                    src_ref=sendb_ref,
                    dst_ref=recv_ref.at[slot],
                    send_sem=send_sems.at[slot],
                    recv_sem=recv_sems.at[slot],
                    device_id=(partner,),
                    device_id_type=pl.DeviceIdType.MESH,
                )
                rdma.start()
                rdma.wait()
                acc_ref[...] += recv_ref[slot].astype(jnp.float32)

            h = jnp.maximum(acc_ref[...], 0.0).astype(jnp.bfloat16)
            w_out = wouts[l][...].astype(jnp.bfloat16)
            xv = jnp.dot(h, w_out, preferred_element_type=jnp.float32)
            if l == N_LAYERS - 1:
                out_ref[...] = xv
            else:
                xv = xv.astype(jnp.bfloat16)

    return pl.pallas_call(
        body,
        out_shape=jax.ShapeDtypeStruct((b, d_shard), jnp.float32),
        in_specs=[pl.BlockSpec(memory_space=pltpu.VMEM)] * 7,
        out_specs=pl.BlockSpec(memory_space=pltpu.VMEM),
        scratch_shapes=[
            pltpu.VMEM((b, hidden), jnp.float32),
            pltpu.VMEM((b, hidden), jnp.bfloat16),
            pltpu.VMEM((N_SLOTS, b, hidden), jnp.bfloat16),
            pltpu.SemaphoreType.DMA((N_SLOTS,)),
            pltpu.SemaphoreType.DMA((N_SLOTS,)),
        ],
        compiler_params=pltpu.CompilerParams(
            collective_id=0,
            vmem_limit_bytes=100 * 1024 * 1024,
        ),
    )(x, Win0, Wout0, Win1, Wout1, Win2, Wout2)


# device time: 79706 ns/iter; 1.6852x vs baseline; 1.1552x over previous
import jax
import jax.numpy as jnp
from jax import lax
from jax.experimental import pallas as pl
from jax.experimental.pallas import tpu as pltpu

N_DEV = 16
LOG2P = 4
N_LAYERS = 3
N_CHUNKS = 2
N_SLOTS = N_LAYERS * N_CHUNKS * LOG2P


def kernel(x, Win0, Wout0, Win1, Wout1, Win2, Wout2):
    b, d_shard = x.shape
    hidden = Win0.shape[1]
    hc = hidden // N_CHUNKS

    def body(x_ref, win0_ref, wout0_ref, win1_ref, wout1_ref, win2_ref,
             wout2_ref, out_ref, acc_ref, sendb_ref, recv_ref, send_sems,
             recv_sems):
        my = lax.axis_index("i")

        barrier_sem = pltpu.get_barrier_semaphore()
        for k in range(LOG2P):
            partner = my ^ (1 << k)
            pl.semaphore_signal(
                barrier_sem, inc=1,
                device_id=(partner,), device_id_type=pl.DeviceIdType.MESH,
            )
        pl.semaphore_wait(barrier_sem, LOG2P)

        wins = [win0_ref, win1_ref, win2_ref]
        wouts = [wout0_ref, wout1_ref, wout2_ref]

        def slot(l, c, k):
            return (l * N_CHUNKS + c) * LOG2P + k

        def start(l, c, k):
            partner = my ^ (1 << k)
            s = slot(l, c, k)
            sendb_ref[c] = acc_ref[c].astype(jnp.bfloat16)
            rdma = pltpu.make_async_remote_copy(
                src_ref=sendb_ref.at[c],
                dst_ref=recv_ref.at[s],
                send_sem=send_sems.at[s],
                recv_sem=recv_sems.at[s],
                device_id=(partner,),
                device_id_type=pl.DeviceIdType.MESH,
            )
            rdma.start()
            return rdma

        xv = x_ref[...].astype(jnp.bfloat16)
        for l in range(N_LAYERS):
            w_in = wins[l]
            w_out = wouts[l]
            inflight = {}

            acc_ref[0] = jnp.dot(xv, w_in[:, 0:hc].astype(jnp.bfloat16),
                                 preferred_element_type=jnp.float32)
            inflight[0] = start(l, 0, 0)
            acc_ref[1] = jnp.dot(xv, w_in[:, hc:hidden].astype(jnp.bfloat16),
                                 preferred_element_type=jnp.float32)
            inflight[1] = start(l, 1, 0)

            yv = None
            for k in range(LOG2P):
                for c in range(N_CHUNKS):
                    inflight[c].wait()
                    acc_ref[c] += recv_ref[slot(l, c, k)].astype(jnp.float32)
                    if k < LOG2P - 1:
                        inflight[c] = start(l, c, k + 1)
                    else:
                        h = jnp.maximum(acc_ref[c], 0.0).astype(jnp.bfloat16)
                        wo = w_out[c * hc:(c + 1) * hc, :].astype(jnp.bfloat16)
                        part = jnp.dot(h, wo,
                                       preferred_element_type=jnp.float32)
                        yv = part if yv is None else yv + part

            if l == N_LAYERS - 1:
                out_ref[...] = yv
            else:
                xv = yv.astype(jnp.bfloat16)

    return pl.pallas_call(
        body,
        out_shape=jax.ShapeDtypeStruct((b, d_shard), jnp.float32),
        in_specs=[pl.BlockSpec(memory_space=pltpu.VMEM)] * 7,
        out_specs=pl.BlockSpec(memory_space=pltpu.VMEM),
        scratch_shapes=[
            pltpu.VMEM((N_CHUNKS, b, hc), jnp.float32),
            pltpu.VMEM((N_CHUNKS, b, hc), jnp.bfloat16),
            pltpu.VMEM((N_SLOTS, b, hc), jnp.bfloat16),
            pltpu.SemaphoreType.DMA((N_SLOTS,)),
            pltpu.SemaphoreType.DMA((N_SLOTS,)),
        ],
        compiler_params=pltpu.CompilerParams(
            collective_id=0,
            vmem_limit_bytes=100 * 1024 * 1024,
        ),
    )(x, Win0, Wout0, Win1, Wout1, Win2, Wout2)


# device time: 61050 ns/iter; 2.2001x vs baseline; 1.3056x over previous
import jax
import jax.numpy as jnp
from jax import lax
from jax.experimental import pallas as pl
from jax.experimental.pallas import tpu as pltpu

N_DEV = 16
N_LAYERS = 3


def kernel(x, Win0, Wout0, Win1, Wout1, Win2, Wout2):
    b, d_shard = x.shape
    hidden = Win0.shape[1]
    hs = hidden // N_DEV

    def body(x_ref, win0_ref, wout0_ref, win1_ref, wout1_ref, win2_ref,
             wout2_ref, out_ref, acc_ref, sendb_ref, rs_recv_ref, hb_ref,
             h_full_ref, loc_sem, rs_send_sems, rs_recv_sems, ag_send_sems,
             ag_recv_sems):
        my = lax.axis_index("i")

        barrier_sem = pltpu.get_barrier_semaphore()
        for o in range(1, N_DEV):
            pl.semaphore_signal(
                barrier_sem, inc=1,
                device_id=((my + o) % N_DEV,),
                device_id_type=pl.DeviceIdType.MESH,
            )
        pl.semaphore_wait(barrier_sem, N_DEV - 1)

        wins = [win0_ref, win1_ref, win2_ref]
        wouts = [wout0_ref, wout1_ref, wout2_ref]

        xv = x_ref[...].astype(jnp.bfloat16)
        for l in range(N_LAYERS):
            pending_sends = []

            acc_ref[...] = jnp.dot(xv, wins[l][...].astype(jnp.bfloat16),
                                   preferred_element_type=jnp.float32)
            sendb_ref[...] = acc_ref[...].astype(jnp.bfloat16)

            for o in range(1, N_DEV):
                p = (my + o) % N_DEV
                rdma = pltpu.make_async_remote_copy(
                    src_ref=sendb_ref.at[:, pl.ds(p * hs, hs)],
                    dst_ref=rs_recv_ref.at[l, my],
                    send_sem=rs_send_sems.at[l, p],
                    recv_sem=rs_recv_sems.at[l, my],
                    device_id=(p,),
                    device_id_type=pl.DeviceIdType.MESH,
                )
                rdma.start()
                pending_sends.append(rdma)
            own = pltpu.make_async_copy(
                sendb_ref.at[:, pl.ds(my * hs, hs)],
                rs_recv_ref.at[l, my],
                loc_sem,
            )
            own.start()
            own.wait()

            red = rs_recv_ref[l, my].astype(jnp.float32)
            for o in range(1, N_DEV):
                p = (my - o) % N_DEV
                wr = pltpu.make_async_remote_copy(
                    src_ref=sendb_ref.at[:, pl.ds(p * hs, hs)],
                    dst_ref=rs_recv_ref.at[l, p],
                    send_sem=rs_send_sems.at[l, p],
                    recv_sem=rs_recv_sems.at[l, p],
                    device_id=(p,),
                    device_id_type=pl.DeviceIdType.MESH,
                )
                wr.wait_recv()
                red += rs_recv_ref[l, p].astype(jnp.float32)

            hb_ref[l] = jnp.maximum(red, 0.0).astype(jnp.bfloat16)

            for o in range(1, N_DEV):
                p = (my + o) % N_DEV
                rdma = pltpu.make_async_remote_copy(
                    src_ref=hb_ref.at[l],
                    dst_ref=h_full_ref.at[l].at[:, pl.ds(my * hs, hs)],
                    send_sem=ag_send_sems.at[l, p],
                    recv_sem=ag_recv_sems.at[l, my],
                    device_id=(p,),
                    device_id_type=pl.DeviceIdType.MESH,
                )
                rdma.start()
                pending_sends.append(rdma)
            own = pltpu.make_async_copy(
                hb_ref.at[l],
                h_full_ref.at[l].at[:, pl.ds(my * hs, hs)],
                loc_sem,
            )
            own.start()
            own.wait()
            for o in range(1, N_DEV):
                p = (my - o) % N_DEV
                wr = pltpu.make_async_remote_copy(
                    src_ref=hb_ref.at[l],
                    dst_ref=h_full_ref.at[l].at[:, pl.ds(p * hs, hs)],
                    send_sem=ag_send_sems.at[l, p],
                    recv_sem=ag_recv_sems.at[l, p],
                    device_id=(p,),
                    device_id_type=pl.DeviceIdType.MESH,
                )
                wr.wait_recv()

            yv = jnp.dot(h_full_ref[l], wouts[l][...].astype(jnp.bfloat16),
                         preferred_element_type=jnp.float32)
            if l == N_LAYERS - 1:
                out_ref[...] = yv
            else:
                xv = yv.astype(jnp.bfloat16)

            for rdma in pending_sends:
                rdma.wait_send()

    return pl.pallas_call(
        body,
        out_shape=jax.ShapeDtypeStruct((b, d_shard), jnp.float32),
        in_specs=[pl.BlockSpec(memory_space=pltpu.VMEM)] * 7,
        out_specs=pl.BlockSpec(memory_space=pltpu.VMEM),
        scratch_shapes=[
            pltpu.VMEM((b, hidden), jnp.float32),
            pltpu.VMEM((b, hidden), jnp.bfloat16),
            pltpu.VMEM((N_LAYERS, N_DEV, b, hs), jnp.bfloat16),
            pltpu.VMEM((N_LAYERS, b, hs), jnp.bfloat16),
            pltpu.VMEM((N_LAYERS, b, hidden), jnp.bfloat16),
            pltpu.SemaphoreType.DMA,
            pltpu.SemaphoreType.DMA((N_LAYERS, N_DEV)),
            pltpu.SemaphoreType.DMA((N_LAYERS, N_DEV)),
            pltpu.SemaphoreType.DMA((N_LAYERS, N_DEV)),
            pltpu.SemaphoreType.DMA((N_LAYERS, N_DEV)),
        ],
        compiler_params=pltpu.CompilerParams(
            collective_id=0,
            vmem_limit_bytes=100 * 1024 * 1024,
        ),
    )(x, Win0, Wout0, Win1, Wout1, Win2, Wout2)


# device time: 59583 ns/iter; 2.2543x vs baseline; 1.0246x over previous
import jax
import jax.numpy as jnp
from jax import lax
from jax.experimental import pallas as pl
from jax.experimental.pallas import tpu as pltpu

N_DEV = 16
N_LAYERS = 3
N_HALF = 2
N_GROUPS = 4


def kernel(x, Win0, Wout0, Win1, Wout1, Win2, Wout2):
    b, d_shard = x.shape
    hidden = Win0.shape[1]
    hs = hidden // N_DEV
    per_half = N_DEV // N_HALF
    per_grp = N_DEV // N_GROUPS

    def body(x_ref, win0_ref, wout0_ref, win1_ref, wout1_ref, win2_ref,
             wout2_ref, out_ref, sendb_ref, rs_recv_ref, hb_ref,
             h_full_ref, rs_send_sems, rs_recv_sems, ag_send_sems,
             ag_recv_sems):
        my = lax.axis_index("i")

        barrier_sem = pltpu.get_barrier_semaphore()
        for o in range(1, N_DEV):
            pl.semaphore_signal(
                barrier_sem, inc=1,
                device_id=((my + o) % N_DEV,),
                device_id_type=pl.DeviceIdType.MESH,
            )
        pl.semaphore_wait(barrier_sem, N_DEV - 1)

        wins = [win0_ref, win1_ref, win2_ref]
        wouts = [wout0_ref, wout1_ref, wout2_ref]

        xv = x_ref[...].astype(jnp.bfloat16)
        for l in range(N_LAYERS):
            for half in range(N_HALF):
                c0 = half * per_half * hs
                c1 = c0 + per_half * hs
                sendb_ref[:, c0:c1] = jnp.dot(
                    xv, wins[l][:, c0:c1].astype(jnp.bfloat16),
                    preferred_element_type=jnp.float32).astype(jnp.bfloat16)
                for p in range(half * per_half, (half + 1) * per_half):
                    @pl.when(my != p)
                    def _():
                        pltpu.make_async_remote_copy(
                            src_ref=sendb_ref.at[:, pl.ds(p * hs, hs)],
                            dst_ref=rs_recv_ref.at[l, my],
                            send_sem=rs_send_sems.at[l, p],
                            recv_sem=rs_recv_sems.at[l, my],
                            device_id=(p,),
                            device_id_type=pl.DeviceIdType.MESH,
                        ).start()

                    @pl.when(my == p)
                    def _():
                        pltpu.make_async_copy(
                            sendb_ref.at[:, pl.ds(p * hs, hs)],
                            rs_recv_ref.at[l, p],
                            rs_recv_sems.at[l, p],
                        ).start()

            red = None
            for o in range(N_DEV):
                p = (my - o) % N_DEV
                pltpu.make_async_copy(
                    rs_recv_ref.at[l, p],
                    rs_recv_ref.at[l, p],
                    rs_recv_sems.at[l, p],
                ).wait()
                s = rs_recv_ref[l, p].astype(jnp.float32)
                red = s if red is None else red + s

            hb_ref[l] = jnp.maximum(red, 0.0).astype(jnp.bfloat16)

            for o in range(1, N_DEV):
                p = (my + o) % N_DEV
                pltpu.make_async_remote_copy(
                    src_ref=hb_ref.at[l],
                    dst_ref=h_full_ref.at[l].at[:, pl.ds(my * hs, hs)],
                    send_sem=ag_send_sems.at[l, p],
                    recv_sem=ag_recv_sems.at[l, my],
                    device_id=(p,),
                    device_id_type=pl.DeviceIdType.MESH,
                ).start()
            pltpu.make_async_copy(
                hb_ref.at[l],
                h_full_ref.at[l].at[:, pl.ds(my * hs, hs)],
                ag_recv_sems.at[l, my],
            ).start()

            yv = None
            for g in range(N_GROUPS):
                for p in range(g * per_grp, (g + 1) * per_grp):
                    pltpu.make_async_copy(
                        h_full_ref.at[l].at[:, pl.ds(p * hs, hs)],
                        h_full_ref.at[l].at[:, pl.ds(p * hs, hs)],
                        ag_recv_sems.at[l, p],
                    ).wait()
                r0 = g * per_grp * hs
                r1 = r0 + per_grp * hs
                part = jnp.dot(
                    h_full_ref[l][:, r0:r1],
                    wouts[l][r0:r1, :].astype(jnp.bfloat16),
                    preferred_element_type=jnp.float32)
                yv = part if yv is None else yv + part

            if l == N_LAYERS - 1:
                out_ref[...] = yv
            else:
                xv = yv.astype(jnp.bfloat16)

            for p in range(N_DEV):
                @pl.when(my != p)
                def _():
                    pltpu.make_async_remote_copy(
                        src_ref=sendb_ref.at[:, pl.ds(p * hs, hs)],
                        dst_ref=rs_recv_ref.at[l, my],
                        send_sem=rs_send_sems.at[l, p],
                        recv_sem=rs_recv_sems.at[l, my],
                        device_id=(p,),
                        device_id_type=pl.DeviceIdType.MESH,
                    ).wait_send()
            for o in range(1, N_DEV):
                p = (my + o) % N_DEV
                pltpu.make_async_remote_copy(
                    src_ref=hb_ref.at[l],
                    dst_ref=h_full_ref.at[l].at[:, pl.ds(my * hs, hs)],
                    send_sem=ag_send_sems.at[l, p],
                    recv_sem=ag_recv_sems.at[l, my],
                    device_id=(p,),
                    device_id_type=pl.DeviceIdType.MESH,
                ).wait_send()

    return pl.pallas_call(
        body,
        out_shape=jax.ShapeDtypeStruct((b, d_shard), jnp.float32),
        in_specs=[pl.BlockSpec(memory_space=pltpu.VMEM)] * 7,
        out_specs=pl.BlockSpec(memory_space=pltpu.VMEM),
        scratch_shapes=[
            pltpu.VMEM((b, hidden), jnp.bfloat16),
            pltpu.VMEM((N_LAYERS, N_DEV, b, hs), jnp.bfloat16),
            pltpu.VMEM((N_LAYERS, b, hs), jnp.bfloat16),
            pltpu.VMEM((N_LAYERS, b, hidden), jnp.bfloat16),
            pltpu.SemaphoreType.DMA((N_LAYERS, N_DEV)),
            pltpu.SemaphoreType.DMA((N_LAYERS, N_DEV)),
            pltpu.SemaphoreType.DMA((N_LAYERS, N_DEV)),
            pltpu.SemaphoreType.DMA((N_LAYERS, N_DEV)),
        ],
        compiler_params=pltpu.CompilerParams(
            collective_id=0,
            vmem_limit_bytes=100 * 1024 * 1024,
        ),
    )(x, Win0, Wout0, Win1, Wout1, Win2, Wout2)
